# baseline (speedup 1.0000x reference)
import functools

import jax
import jax.numpy as jnp
from jax import lax
from jax.experimental import pallas as pl
from jax.experimental.pallas import tpu as pltpu
from jax.experimental.pallas import tpu_sc as plsc

D = H = W = 128
DHW = D * H * W
N = 4
V = 100000
F = 200000
F3 = 3 * F

NTILES = 16
ROUND = 1568
WIN = NTILES * ROUND
NWIN = 4

PER = 37504
FSTAGE = 2048
NFULL = 18
FTAIL = PER - NFULL * FSTAGE

ZCHUNK = 2048
ZITERS = (2 * DHW // NTILES) // ZCHUNK

MAGIC = 8388608.0

_sc_mesh = plsc.VectorSubcoreMesh(core_axis_name="c", subcore_axis_name="s")


@functools.partial(
    pl.kernel,
    out_type=jax.ShapeDtypeStruct((N * DHW,), jnp.float32),
    mesh=_sc_mesh,
    compiler_params=pltpu.CompilerParams(needs_layout_passes=False),
    scratch_types=[
        pltpu.VMEM((ROUND * 3,), jnp.float32),
        pltpu.VMEM((ROUND,), jnp.int32),
        pltpu.VMEM((WIN,), jnp.int32),
        pltpu.VMEM((N, FSTAGE), jnp.int32),
        pltpu.VMEM((ROUND,), jnp.int32),
        pltpu.VMEM((ROUND,), jnp.int32),
        pltpu.VMEM((ROUND,), jnp.int32),
        pltpu.VMEM((ROUND,), jnp.float32),
        pltpu.VMEM((ZCHUNK,), jnp.float32),
        pltpu.VMEM_SHARED((NTILES * WIN,), jnp.int32),
        pltpu.SemaphoreType.DMA,
    ],
)
def _raster(verts_hbm, faces_hbm, vol_hbm, vbuf_v, oid_v, lmask_v, fidx_v,
            tmp_v, acc_v, sid_v, ones_v, zf_v, smask_sh, sem):
    c = lax.axis_index("c")
    s = lax.axis_index("s")

    zero16f = jnp.zeros((16,), jnp.float32)
    zero16i = jnp.zeros((16,), jnp.int32)
    one16f = jnp.ones((16,), jnp.float32)
    one16i = jnp.ones((16,), jnp.int32)
    lane = lax.iota(jnp.int32, 16)

    def _zf(i, carry):
        zf_v[pl.ds(i * 16, 16)] = zero16f
        return carry

    lax.fori_loop(0, ZCHUNK // 16, _zf, 0)

    def _of(i, carry):
        ones_v[pl.ds(i * 16, 16)] = one16f
        return carry

    lax.fori_loop(0, ROUND // 16, _of, 0)

    zbase = c * (2 * DHW) + s * (2 * DHW // NTILES)

    def _zvol(i, carry):
        pltpu.sync_copy(zf_v, vol_hbm.at[pl.ds(zbase + i * ZCHUNK, ZCHUNK)])
        return carry

    lax.fori_loop(0, ZITERS, _zvol, 0)

    fbase = pl.multiple_of(s * PER, 128)

    for bb in range(2):
        b = c * 2 + bb
        vol_off = b * DHW

        for p in range(NWIN):
            vb = pl.multiple_of(
                jnp.minimum(p * WIN + s * ROUND, V - ROUND), 8)

            def _zl(i, carry):
                lmask_v[pl.ds(i * 16, 16)] = zero16i
                return carry

            lax.fori_loop(0, WIN // 16, _zl, 0)

            voff = pl.multiple_of((b * V + vb) * 3, 8)
            pltpu.sync_copy(verts_hbm.at[pl.ds(voff, ROUND * 3)], vbuf_v)

            def _quant(g, carry):
                row3 = (lane + g * 16) * 3
                zc = plsc.load_gather(vbuf_v, [row3])
                yc = plsc.load_gather(vbuf_v, [row3 + 1])
                xc = plsc.load_gather(vbuf_v, [row3 + 2])

                def to_vox(t):
                    r = (t + 1.0) * 63.5
                    r = (r + MAGIC) - MAGIC
                    return jnp.clip(r.astype(jnp.int32), 0, D - 1)

                flat = (to_vox(zc) * H + to_vox(yc)) * W + to_vox(xc)
                oid_v[pl.ds(g * 16, 16)] = flat + vol_off
                return carry

            lax.fori_loop(0, ROUND // 16, _quant, 0)

            def _mark(g, carry):
                idx = fidx_v[b, pl.ds(g * 16, 16)]
                w = idx - p * WIN
                m = (w >= 0) & (w < WIN)
                plsc.store_scatter(lmask_v, [w], one16i, mask=m)
                return carry

            def _fstage(st, carry):
                pltpu.sync_copy(
                    faces_hbm.at[:, pl.ds(fbase + st * FSTAGE, FSTAGE)],
                    fidx_v)
                lax.fori_loop(0, FSTAGE // 16, _mark, 0)
                return carry

            lax.fori_loop(0, NFULL, _fstage, 0)
            pltpu.sync_copy(
                faces_hbm.at[:, pl.ds(fbase + NFULL * FSTAGE, FTAIL)],
                fidx_v.at[:, pl.ds(0, FTAIL)])

            def _mark_tail(g, carry):
                idx = fidx_v[b, pl.ds(g * 16, 16)]
                w = idx - p * WIN
                pos = fbase + NFULL * FSTAGE + g * 16 + lane
                m = (w >= 0) & (w < WIN) & (pos < F3)
                plsc.store_scatter(lmask_v, [w], one16i, mask=m)
                return carry

            lax.fori_loop(0, FTAIL // 16, _mark_tail, 0)

            pltpu.sync_copy(lmask_v, smask_sh.at[pl.ds(s * WIN, WIN)])
            plsc.subcore_barrier()

            woff = pl.multiple_of(vb - p * WIN, 8)
            for t in range(NTILES):
                pltpu.sync_copy(
                    smask_sh.at[pl.ds(t * WIN + woff, ROUND)], tmp_v)

                def _accum(g, carry):
                    chunk = tmp_v[pl.ds(g * 16, 16)]
                    if t == 0:
                        acc_v[pl.ds(g * 16, 16)] = chunk
                    else:
                        acc_v[pl.ds(g * 16, 16)] = (
                            acc_v[pl.ds(g * 16, 16)] | chunk)
                    return carry

                lax.fori_loop(0, ROUND // 16, _accum, 0)

            def _padmax(g, pad):
                m = acc_v[pl.ds(g * 16, 16)] > 0
                gid = oid_v[pl.ds(g * 16, 16)]
                return jnp.maximum(pad, jnp.where(m, gid, -1))

            pad = lax.fori_loop(0, ROUND // 16, _padmax,
                                jnp.full((16,), -1, jnp.int32))
            pad_s = lax.reduce_max(pad, axes=(0,))
            pad_v = jnp.full((16,), 1, jnp.int32) * pad_s

            def _sel(g, carry):
                m = acc_v[pl.ds(g * 16, 16)] > 0
                gid = oid_v[pl.ds(g * 16, 16)]
                sid_v[pl.ds(g * 16, 16)] = jnp.where(m, gid, pad_v)
                return carry

            lax.fori_loop(0, ROUND // 16, _sel, 0)

            @pl.when(pad_s >= 0)
            def _fire():
                pltpu.async_copy(ones_v, vol_hbm.at[sid_v], sem).wait()

            plsc.subcore_barrier()


def kernel(vertices, faces):
    vol = _raster(vertices.reshape(-1), faces.reshape(N, F3))
    return vol.reshape(N, D, H, W)


# speedup vs baseline: 1.0166x; 1.0166x over previous
import functools

import jax
import jax.numpy as jnp
from jax import lax
from jax.experimental import pallas as pl
from jax.experimental.pallas import tpu as pltpu
from jax.experimental.pallas import tpu_sc as plsc

D = H = W = 128
DHW = D * H * W
N = 4
V = 100000
F = 200000
F3 = 3 * F

NTILES = 16
ROUND = 1568
WIN = NTILES * ROUND
NWIN = 4

PER = 37504
FLAST = F3 - PER
FSTAGE = 2048
NFULL = 18
FTAIL = PER - NFULL * FSTAGE

ZCHUNK = 2048
ZITERS = (2 * DHW // NTILES) // ZCHUNK

MAGIC = 8388608.0

_sc_mesh = plsc.VectorSubcoreMesh(core_axis_name="c", subcore_axis_name="s")


@functools.partial(
    pl.kernel,
    out_type=jax.ShapeDtypeStruct((N * DHW,), jnp.float32),
    mesh=_sc_mesh,
    compiler_params=pltpu.CompilerParams(needs_layout_passes=False),
    scratch_types=[
        pltpu.VMEM((ROUND * 3,), jnp.float32),
        pltpu.VMEM((ROUND,), jnp.int32),
        pltpu.VMEM((WIN,), jnp.int32),
        pltpu.VMEM((FSTAGE,), jnp.int32),
        pltpu.VMEM((ROUND,), jnp.int32),
        pltpu.VMEM((ROUND,), jnp.int32),
        pltpu.VMEM((ROUND,), jnp.int32),
        pltpu.VMEM((ROUND,), jnp.float32),
        pltpu.VMEM((ZCHUNK,), jnp.float32),
        pltpu.VMEM_SHARED((NTILES * WIN,), jnp.int32),
        pltpu.SemaphoreType.DMA,
    ],
)
def _raster(verts_hbm, faces_hbm, vol_hbm, vbuf_v, oid_v, lmask_v, fidx_v,
            tmp_v, acc_v, sid_v, ones_v, zf_v, smask_sh, sem):
    c = lax.axis_index("c")
    s = lax.axis_index("s")

    zero16f = jnp.zeros((16,), jnp.float32)
    zero16i = jnp.zeros((16,), jnp.int32)
    one16f = jnp.ones((16,), jnp.float32)
    one16i = jnp.ones((16,), jnp.int32)
    lane = lax.iota(jnp.int32, 16)

    def _zf(i, carry):
        zf_v[pl.ds(i * 16, 16)] = zero16f
        return carry

    lax.fori_loop(0, ZCHUNK // 16, _zf, 0)

    def _of(i, carry):
        ones_v[pl.ds(i * 16, 16)] = one16f
        return carry

    lax.fori_loop(0, ROUND // 16, _of, 0)

    zbase = c * (2 * DHW) + s * (2 * DHW // NTILES)

    def _zvol(i, carry):
        pltpu.sync_copy(zf_v, vol_hbm.at[pl.ds(zbase + i * ZCHUNK, ZCHUNK)])
        return carry

    lax.fori_loop(0, ZITERS, _zvol, 0)

    fbase = pl.multiple_of(jnp.where(s == NTILES - 1, FLAST, s * PER), 64)

    for bb in range(2):
        b = c * 2 + bb
        vol_off = b * DHW

        for p in range(NWIN):
            vb = pl.multiple_of(
                jnp.minimum(p * WIN + s * ROUND, V - ROUND), 8)

            def _zl(i, carry):
                lmask_v[pl.ds(i * 16, 16)] = zero16i
                return carry

            lax.fori_loop(0, WIN // 16, _zl, 0)

            voff = pl.multiple_of((b * V + vb) * 3, 8)
            pltpu.sync_copy(verts_hbm.at[pl.ds(voff, ROUND * 3)], vbuf_v)

            def _quant(g, carry):
                row3 = (lane + g * 16) * 3
                zc = plsc.load_gather(vbuf_v, [row3])
                yc = plsc.load_gather(vbuf_v, [row3 + 1])
                xc = plsc.load_gather(vbuf_v, [row3 + 2])

                def to_vox(t):
                    r = (t + 1.0) * 63.5
                    r = (r + MAGIC) - MAGIC
                    return jnp.clip(r.astype(jnp.int32), 0, D - 1)

                flat = (to_vox(zc) * H + to_vox(yc)) * W + to_vox(xc)
                oid_v[pl.ds(g * 16, 16)] = flat + vol_off
                return carry

            lax.fori_loop(0, ROUND // 16, _quant, 0)

            def _mark_groups(ngroups):
                def _mark(g, carry):
                    idx = fidx_v[pl.ds(g * 16, 16)]
                    w = idx - p * WIN
                    m = (w >= 0) & (w < WIN)
                    plsc.store_scatter(lmask_v, [w], one16i, mask=m)
                    return carry

                lax.fori_loop(0, ngroups, _mark, 0)

            def _fstage(st, carry):
                pltpu.sync_copy(
                    faces_hbm.at[pl.ds(b * F3 + fbase + st * FSTAGE,
                                       FSTAGE)], fidx_v)
                _mark_groups(FSTAGE // 16)
                return carry

            lax.fori_loop(0, NFULL, _fstage, 0)
            pltpu.sync_copy(
                faces_hbm.at[pl.ds(b * F3 + fbase + NFULL * FSTAGE, FTAIL)],
                fidx_v.at[pl.ds(0, FTAIL)])
            _mark_groups(FTAIL // 16)

            pltpu.sync_copy(lmask_v, smask_sh.at[pl.ds(s * WIN, WIN)])
            plsc.subcore_barrier()

            woff = pl.multiple_of(vb - p * WIN, 8)
            for t in range(NTILES):
                pltpu.sync_copy(
                    smask_sh.at[pl.ds(t * WIN + woff, ROUND)], tmp_v)

                def _accum(g, carry):
                    chunk = tmp_v[pl.ds(g * 16, 16)]
                    if t == 0:
                        acc_v[pl.ds(g * 16, 16)] = chunk
                    else:
                        acc_v[pl.ds(g * 16, 16)] = (
                            acc_v[pl.ds(g * 16, 16)] | chunk)
                    return carry

                lax.fori_loop(0, ROUND // 16, _accum, 0)

            def _padmax(g, pad):
                m = acc_v[pl.ds(g * 16, 16)] > 0
                gid = oid_v[pl.ds(g * 16, 16)]
                return jnp.maximum(pad, jnp.where(m, gid, -1))

            pad = lax.fori_loop(0, ROUND // 16, _padmax,
                                jnp.full((16,), -1, jnp.int32))
            pad_s = lax.reduce_max(pad, axes=(0,))
            pad_v = jnp.full((16,), 1, jnp.int32) * pad_s

            def _sel(g, carry):
                m = acc_v[pl.ds(g * 16, 16)] > 0
                gid = oid_v[pl.ds(g * 16, 16)]
                sid_v[pl.ds(g * 16, 16)] = jnp.where(m, gid, pad_v)
                return carry

            lax.fori_loop(0, ROUND // 16, _sel, 0)

            @pl.when(pad_s >= 0)
            def _fire():
                pltpu.async_copy(ones_v, vol_hbm.at[sid_v], sem).wait()

            plsc.subcore_barrier()


def kernel(vertices, faces):
    vol = _raster(vertices.reshape(-1), faces.reshape(-1))
    return vol.reshape(N, D, H, W)
